# baseline (speedup 1.0000x reference)
import functools

import jax
import jax.numpy as jnp
from jax import lax
from jax.experimental import pallas as pl
from jax.experimental.pallas import tpu as pltpu
from jax.experimental.pallas import tpu_sc as plsc


def _proj_body(tab_ref, w_ref, b_ref, out_ref):
    out_ref[...] = lax.dot_general(
        tab_ref[...], w_ref[...],
        dimension_numbers=(((1,), (1,)), ((), ())),
        preferred_element_type=jnp.float32,
    ) + b_ref[...]


def _project_table(table, W, b):
    V, D = table.shape
    R = 4000
    assert V % R == 0
    return pl.pallas_call(
        _proj_body,
        grid=(V // R,),
        in_specs=[
            pl.BlockSpec((R, D), lambda i: (i, 0)),
            pl.BlockSpec((D, D), lambda i: (0, 0)),
            pl.BlockSpec((1, D), lambda i: (0, 0)),
        ],
        out_specs=pl.BlockSpec((R, D), lambda i: (i, 0)),
        out_shape=jax.ShapeDtypeStruct((V, D), jnp.float32),
    )(table, W, b.reshape(1, D))


_NW = 8


def _make_gather(V, D, BATCH, HIST, nc, ns):
    nw = nc * ns
    wb2 = BATCH // 2
    assert wb2 % (nw * _NW) == 0
    w_per_worker = wb2 // nw
    nch_half = w_per_worker // _NW
    nch = 2 * nch_half

    def body(tab_hbm, ids_e, ids_o, y_hbm, idx_v, rows_v,
             gsem0, gsem1, ssem0, ssem1):
        wid = lax.axis_index("s") * nc + lax.axis_index("c")
        wbase = wid * w_per_worker
        gsems = (gsem0, gsem1)
        ssems = (ssem0, ssem1)
        ids_hbm = (ids_e, ids_o)

        def col0(half):
            return wbase + half * _NW

        def fire_gathers(buf, gsem):
            for l in range(HIST):
                pltpu.async_copy(
                    tab_hbm.at[idx_v.at[buf, l]],
                    rows_v.at[buf, l],
                    gsem,
                )

        def drain_gather(buf):
            pltpu.make_async_copy(
                y_hbm.at[:, pl.ds(0, _NW), pl.ds(0, D)],
                rows_v.at[buf],
                gsems[buf],
            ).wait()

        def drain_scatter(buf):
            pltpu.make_async_copy(
                rows_v.at[buf],
                y_hbm.at[:, pl.ds(0, _NW), pl.ds(0, D)],
                ssems[buf],
            ).wait()

        def stage_idx(par, half, buf):
            pltpu.sync_copy(
                ids_hbm[par].at[:, pl.ds(col0(half), _NW)], idx_v.at[buf]
            )

        def scatter(par, half, buf, ssem):
            pltpu.async_copy(
                rows_v.at[buf],
                y_hbm.at[:, pl.ds(col0(half), _NW), pl.ds(par * D, D)],
                ssem,
            )

        stage_idx(0, 0, 0)
        fire_gathers(0, gsems[0])

        def pair(gg, carry):
            for par in (0, 1):
                g = 2 * gg + par
                nbuf = 1 - par

                @pl.when(g + 1 < nch)
                def _():
                    @pl.when(g >= 1)
                    def _():
                        drain_scatter(nbuf)

                    stage_idx(1 - par, gg + par, nbuf)
                    fire_gathers(nbuf, gsems[nbuf])

                drain_gather(par)
                scatter(par, gg, par, ssems[par])
            return carry

        lax.fori_loop(0, nch // 2, pair, 0)
        drain_scatter(0)
        drain_scatter(1)

    return pl.kernel(
        body,
        out_type=jax.ShapeDtypeStruct((HIST, wb2, 2 * D), jnp.float32),
        mesh=plsc.VectorSubcoreMesh(core_axis_name="c", subcore_axis_name="s"),
        scratch_types=[
            pltpu.VMEM((2, HIST, _NW), jnp.int32),
            pltpu.VMEM((2, HIST, _NW, D), jnp.float32),
            pltpu.SemaphoreType.DMA,
            pltpu.SemaphoreType.DMA,
            pltpu.SemaphoreType.DMA,
            pltpu.SemaphoreType.DMA,
        ],
        compiler_params=pltpu.CompilerParams(use_tc_tiling_on_sc=False),
    )


def kernel(emotion_ids, table, W, b):
    BATCH, HIST = emotion_ids.shape
    V, D = table.shape

    table2 = _project_table(table, W, b)

    info = plsc.get_sparse_core_info()
    gather = _make_gather(V, D, BATCH, HIST, info.num_cores, info.num_subcores)
    ids_t = emotion_ids.astype(jnp.int32).T
    y = gather(table2, ids_t[:, 0::2], ids_t[:, 1::2])
    return y.reshape(HIST, BATCH, D).transpose(1, 0, 2)


# speedup vs baseline: 1.2609x; 1.2609x over previous
import functools

import jax
import jax.numpy as jnp
from jax import lax
from jax.experimental import pallas as pl
from jax.experimental.pallas import tpu as pltpu
from jax.experimental.pallas import tpu_sc as plsc


def _proj_body(tab_ref, w_ref, b_ref, out_ref):
    out_ref[...] = lax.dot_general(
        tab_ref[...], w_ref[...],
        dimension_numbers=(((1,), (1,)), ((), ())),
        preferred_element_type=jnp.float32,
    ) + b_ref[...]


def _project_table(table, W, b):
    V, D = table.shape
    R = 4000
    assert V % R == 0
    return pl.pallas_call(
        _proj_body,
        grid=(V // R,),
        in_specs=[
            pl.BlockSpec((R, D), lambda i: (i, 0)),
            pl.BlockSpec((D, D), lambda i: (0, 0)),
            pl.BlockSpec((1, D), lambda i: (0, 0)),
        ],
        out_specs=pl.BlockSpec((R, D), lambda i: (i, 0)),
        out_shape=jax.ShapeDtypeStruct((V, D), jnp.float32),
    )(table, W, b.reshape(1, D))


_NB = 16


def _make_gather(V, D, BATCH, HIST, nc, ns):
    nw = nc * ns
    assert BATCH % (nw * _NB) == 0
    b_per_w = BATCH // nw
    nch = b_per_w // _NB
    assert nch % 2 == 0

    def body(tab_hbm, ids_hbm, out_hbm, idx_v, rows_v, gsem0, gsem1, ssem0, ssem1):
        wid = lax.axis_index("s") * nc + lax.axis_index("c")
        wbase = wid * b_per_w
        gsems = (gsem0, gsem1)
        ssems = (ssem0, ssem1)

        def fire_gathers(buf, gsem):
            for l in range(HIST):
                pltpu.async_copy(
                    tab_hbm.at[idx_v.at[buf, l]],
                    rows_v.at[buf, l],
                    gsem,
                )

        def drain_gather(buf):
            pltpu.make_async_copy(
                out_hbm.at[:, pl.ds(0, _NB), :], rows_v.at[buf], gsems[buf]
            ).wait()

        def drain_scatter(buf):
            pltpu.make_async_copy(
                rows_v.at[buf], out_hbm.at[:, pl.ds(0, _NB), :], ssems[buf]
            ).wait()

        pltpu.sync_copy(ids_hbm.at[:, pl.ds(wbase, _NB)], idx_v.at[0])
        fire_gathers(0, gsems[0])

        def pair(gg, carry):
            for par in (0, 1):
                g = 2 * gg + par
                nbuf = 1 - par

                @pl.when(g + 1 < nch)
                def _():
                    @pl.when(g >= 1)
                    def _():
                        drain_scatter(nbuf)

                    pltpu.sync_copy(
                        ids_hbm.at[:, pl.ds(wbase + (g + 1) * _NB, _NB)],
                        idx_v.at[nbuf],
                    )
                    fire_gathers(nbuf, gsems[nbuf])

                drain_gather(par)
                pltpu.async_copy(
                    rows_v.at[par],
                    out_hbm.at[:, pl.ds(wbase + g * _NB, _NB), :],
                    ssems[par],
                )
            return carry

        lax.fori_loop(0, nch // 2, pair, 0)
        drain_scatter(0)
        drain_scatter(1)

    return pl.kernel(
        body,
        out_type=jax.ShapeDtypeStruct((HIST, BATCH, D), jnp.float32),
        mesh=plsc.VectorSubcoreMesh(core_axis_name="c", subcore_axis_name="s"),
        scratch_types=[
            pltpu.VMEM((2, HIST, _NB), jnp.int32),
            pltpu.VMEM((2, HIST, _NB, D), jnp.float32),
            pltpu.SemaphoreType.DMA,
            pltpu.SemaphoreType.DMA,
            pltpu.SemaphoreType.DMA,
            pltpu.SemaphoreType.DMA,
        ],
        compiler_params=pltpu.CompilerParams(use_tc_tiling_on_sc=False),
    )


def kernel(emotion_ids, table, W, b):
    BATCH, HIST = emotion_ids.shape
    V, D = table.shape

    table2 = _project_table(table, W, b)

    info = plsc.get_sparse_core_info()
    gather = _make_gather(V, D, BATCH, HIST, info.num_cores, info.num_subcores)
    ids_t = emotion_ids.astype(jnp.int32).T
    ot = gather(table2, ids_t)
    return jnp.transpose(ot, (1, 0, 2))
